# speedup vs baseline: 2.4202x; 2.4202x over previous
import functools

import jax
import jax.numpy as jnp
from jax import lax
from jax.experimental import pallas as pl
from jax.experimental.pallas import tpu as pltpu
from jax.experimental.pallas import tpu_sc as plsc

N_NODES = 10000
N_EDGES = 320000
D_FEAT = 128
D_EDGE = 16
D_OUT = 128

NC = 2
NS = 16
NW = NC * NS
E_PER_W = N_EDGES // NW
CHUNK = 80
N_CHUNKS = E_PER_W // CHUNK


def _tables_body(x_ref, w1_ref, w3_ref, b_ref, p1_ref, p3_ref):
    xb = x_ref[...]
    p1_ref[...] = (
        jnp.dot(xb, w1_ref[...], preferred_element_type=jnp.float32)
        + b_ref[...]
    )
    p3_ref[...] = jnp.dot(xb, w3_ref[...], preferred_element_type=jnp.float32)


def _make_tables(x, w1, w3, b2d):
    grid = 5
    blk = N_NODES // grid
    return pl.pallas_call(
        _tables_body,
        grid=(grid,),
        in_specs=[
            pl.BlockSpec((blk, D_FEAT), lambda i: (i, 0)),
            pl.BlockSpec((D_FEAT, D_OUT), lambda i: (0, 0)),
            pl.BlockSpec((D_FEAT, D_OUT), lambda i: (0, 0)),
            pl.BlockSpec((1, D_OUT), lambda i: (0, 0)),
        ],
        out_specs=[
            pl.BlockSpec((blk, D_OUT), lambda i: (i, 0)),
            pl.BlockSpec((blk, D_OUT), lambda i: (i, 0)),
        ],
        out_shape=[
            jax.ShapeDtypeStruct((N_NODES, D_OUT), jnp.float32),
            jax.ShapeDtypeStruct((N_NODES, D_OUT), jnp.float32),
        ],
    )(x, w1, w3, b2d)


def _ea_body(ea_ref, w2_ref, out_ref):
    out_ref[...] = jnp.dot(
        ea_ref[...], w2_ref[...], preferred_element_type=jnp.float32
    )


def _make_ea(edge_attr, w2):
    grid = 40
    blk = N_EDGES // grid
    return pl.pallas_call(
        _ea_body,
        grid=(grid,),
        in_specs=[
            pl.BlockSpec((blk, D_EDGE), lambda i: (i, 0)),
            pl.BlockSpec((D_EDGE, D_OUT), lambda i: (0, 0)),
        ],
        out_specs=pl.BlockSpec((blk, D_OUT), lambda i: (i, 0)),
        out_shape=jax.ShapeDtypeStruct((N_EDGES, D_OUT), jnp.float32),
    )(edge_attr, w2)


def _sc_body(p1_hbm, p3_hbm, src_hbm, dst_hbm, ea_hbm, out_hbm,
             idx1_v, idx3_v, g1_v, g3_v, acc_v, sem):
    wid = lax.axis_index("s") * NC + lax.axis_index("c")
    w_base = wid * E_PER_W

    def chunk_body(c, _):
        base = w_base + c * CHUNK
        pltpu.sync_copy(src_hbm.at[pl.ds(base, CHUNK)], idx1_v)
        pltpu.sync_copy(dst_hbm.at[pl.ds(base, CHUNK)], idx3_v)
        cp1 = pltpu.async_copy(p1_hbm.at[idx1_v], g1_v, sem)
        cp3 = pltpu.async_copy(p3_hbm.at[idx3_v], g3_v, sem)
        pltpu.sync_copy(ea_hbm.at[pl.ds(base, CHUNK)], acc_v)
        cp1.wait()
        cp3.wait()

        def row_body(r, _):
            for j in range(D_OUT // 16):
                sl = pl.ds(j * 16, 16)
                acc_v[r, sl] = acc_v[r, sl] + g1_v[r, sl] + g3_v[r, sl]
            return 0

        lax.fori_loop(0, CHUNK, row_body, 0)
        pltpu.sync_copy(acc_v, out_hbm.at[pl.ds(base, CHUNK)])
        return 0

    lax.fori_loop(0, N_CHUNKS, chunk_body, 0)


def _sc_combine(p1, p3, src, dst, ea):
    mesh = plsc.VectorSubcoreMesh(core_axis_name="c", subcore_axis_name="s")
    f = functools.partial(
        pl.kernel,
        mesh=mesh,
        out_type=jax.ShapeDtypeStruct((N_EDGES, D_OUT), jnp.float32),
        scratch_types=[
            pltpu.VMEM((CHUNK,), jnp.int32),
            pltpu.VMEM((CHUNK,), jnp.int32),
            pltpu.VMEM((CHUNK, D_OUT), jnp.float32),
            pltpu.VMEM((CHUNK, D_OUT), jnp.float32),
            pltpu.VMEM((CHUNK, D_OUT), jnp.float32),
            pltpu.SemaphoreType.DMA,
        ],
    )(_sc_body)
    return f(p1, p3, src, dst, ea)


@jax.jit
def kernel(x, edge_attr, edge_index, W, b):
    w1 = W[:D_FEAT]
    w2 = W[D_FEAT:D_FEAT + D_EDGE]
    w3 = W[D_FEAT + D_EDGE:]
    b2d = b.reshape(1, D_OUT)
    p1, p3 = _make_tables(x, w1, w3, b2d)
    ea = _make_ea(edge_attr, w2)
    return _sc_combine(p1, p3, edge_index[0], edge_index[1], ea)


# speedup vs baseline: 4.0758x; 1.6841x over previous
import functools

import jax
import jax.numpy as jnp
from jax import lax
from jax.experimental import pallas as pl
from jax.experimental.pallas import tpu as pltpu
from jax.experimental.pallas import tpu_sc as plsc

N_NODES = 10000
N_EDGES = 320000
D_FEAT = 128
D_EDGE = 16
D_OUT = 128

NC = 2
NS = 16
NW = NC * NS
E_PER_W = N_EDGES // NW
CHUNK = 80
N_CHUNKS = E_PER_W // CHUNK


def _tables_body(x_ref, w1_ref, w3_ref, b_ref, p1_ref, p3_ref):
    xb = x_ref[...]
    p1_ref[...] = (
        jnp.dot(xb, w1_ref[...], preferred_element_type=jnp.float32)
        + b_ref[...]
    )
    p3_ref[...] = jnp.dot(xb, w3_ref[...], preferred_element_type=jnp.float32)


def _make_tables(x, w1, w3, b2d):
    grid = 5
    blk = N_NODES // grid
    return pl.pallas_call(
        _tables_body,
        grid=(grid,),
        in_specs=[
            pl.BlockSpec((blk, D_FEAT), lambda i: (i, 0)),
            pl.BlockSpec((D_FEAT, D_OUT), lambda i: (0, 0)),
            pl.BlockSpec((D_FEAT, D_OUT), lambda i: (0, 0)),
            pl.BlockSpec((1, D_OUT), lambda i: (0, 0)),
        ],
        out_specs=[
            pl.BlockSpec((blk, D_OUT), lambda i: (i, 0)),
            pl.BlockSpec((blk, D_OUT), lambda i: (i, 0)),
        ],
        out_shape=[
            jax.ShapeDtypeStruct((N_NODES, D_OUT), jnp.float32),
            jax.ShapeDtypeStruct((N_NODES, D_OUT), jnp.float32),
        ],
    )(x, w1, w3, b2d)


def _ea_body(ea_ref, w2_ref, out_ref):
    out_ref[...] = jnp.dot(
        ea_ref[...], w2_ref[...], preferred_element_type=jnp.float32
    )


def _make_ea(edge_attr, w2):
    grid = 40
    blk = N_EDGES // grid
    return pl.pallas_call(
        _ea_body,
        grid=(grid,),
        in_specs=[
            pl.BlockSpec((blk, D_EDGE), lambda i: (i, 0)),
            pl.BlockSpec((D_EDGE, D_OUT), lambda i: (0, 0)),
        ],
        out_specs=pl.BlockSpec((blk, D_OUT), lambda i: (i, 0)),
        out_shape=jax.ShapeDtypeStruct((N_EDGES, D_OUT), jnp.float32),
    )(edge_attr, w2)


def _sc_body(p1_hbm, p3_hbm, src_hbm, dst_hbm, ea_hbm, out_hbm,
             idx1_v, idx3_v,
             g1_0, g1_1, g3_0, g3_1, acc_0, acc_1, ob_0, ob_1,
             gsem0, gsem1, easem0, easem1, osem0, osem1):
    wid = lax.axis_index("s") * NC + lax.axis_index("c")
    w_base = wid * E_PER_W

    g1 = (g1_0, g1_1)
    g3 = (g3_0, g3_1)
    acc = (acc_0, acc_1)
    ob = (ob_0, ob_1)
    gsem = (gsem0, gsem1)
    easem = (easem0, easem1)
    osem = (osem0, osem1)

    pltpu.sync_copy(src_hbm.at[pl.ds(w_base, E_PER_W)], idx1_v)
    pltpu.sync_copy(dst_hbm.at[pl.ds(w_base, E_PER_W)], idx3_v)

    def in_descs(c, b):
        base = w_base + c * CHUNK
        lb = c * CHUNK
        return (
            pltpu.make_async_copy(
                p1_hbm.at[idx1_v.at[pl.ds(lb, CHUNK)]], g1[b], gsem[b]),
            pltpu.make_async_copy(
                p3_hbm.at[idx3_v.at[pl.ds(lb, CHUNK)]], g3[b], gsem[b]),
            pltpu.make_async_copy(
                ea_hbm.at[pl.ds(base, CHUNK)], acc[b], easem[b]),
        )

    def out_desc(c, b):
        base = w_base + c * CHUNK
        return pltpu.make_async_copy(
            ob[b], out_hbm.at[pl.ds(base, CHUNK)], osem[b])

    def issue(c, b):
        for d in in_descs(c, b):
            d.start()

    def wait_in(c, b):
        for d in in_descs(c, b):
            d.wait()

    def compute(b):
        def row_body(r, _):
            for j in range(D_OUT // 16):
                sl = pl.ds(j * 16, 16)
                ob[b][r, sl] = acc[b][r, sl] + g1[b][r, sl] + g3[b][r, sl]
            return 0

        lax.fori_loop(0, CHUNK, row_body, 0)

    def step(c, b, do_wait_out, do_issue_next):
        wait_in(c, b)
        if do_wait_out:
            out_desc(c - 2, b).wait()
        compute(b)
        if do_issue_next:
            issue(c + 2, b)
        out_desc(c, b).start()

    issue(0, 0)
    issue(1, 1)
    step(0, 0, False, True)
    step(1, 1, False, True)

    def pair_body(i, _):
        step(2 * i, 0, True, True)
        step(2 * i + 1, 1, True, True)
        return 0

    lax.fori_loop(1, (N_CHUNKS - 3) // 2, pair_body, 0)

    step(N_CHUNKS - 3, 0, True, True)
    step(N_CHUNKS - 2, 1, True, False)
    step(N_CHUNKS - 1, 0, True, False)
    out_desc(N_CHUNKS - 2, 1).wait()
    out_desc(N_CHUNKS - 1, 0).wait()


def _sc_combine(p1, p3, src, dst, ea):
    mesh = plsc.VectorSubcoreMesh(core_axis_name="c", subcore_axis_name="s")
    blk = lambda: pltpu.VMEM((CHUNK, D_OUT), jnp.float32)
    f = functools.partial(
        pl.kernel,
        mesh=mesh,
        out_type=jax.ShapeDtypeStruct((N_EDGES, D_OUT), jnp.float32),
        scratch_types=[
            pltpu.VMEM((E_PER_W,), jnp.int32),
            pltpu.VMEM((E_PER_W,), jnp.int32),
            blk(), blk(), blk(), blk(), blk(), blk(), blk(), blk(),
            pltpu.SemaphoreType.DMA,
            pltpu.SemaphoreType.DMA,
            pltpu.SemaphoreType.DMA,
            pltpu.SemaphoreType.DMA,
            pltpu.SemaphoreType.DMA,
            pltpu.SemaphoreType.DMA,
        ],
    )(_sc_body)
    return f(p1, p3, src, dst, ea)


@jax.jit
def kernel(x, edge_attr, edge_index, W, b):
    w1 = W[:D_FEAT]
    w2 = W[D_FEAT:D_FEAT + D_EDGE]
    w3 = W[D_FEAT + D_EDGE:]
    b2d = b.reshape(1, D_OUT)
    p1, p3 = _make_tables(x, w1, w3, b2d)
    ea = _make_ea(edge_attr, w2)
    return _sc_combine(p1, p3, edge_index[0], edge_index[1], ea)
